# baseline (speedup 1.0000x reference)
import jax
import jax.numpy as jnp
from jax.experimental import pallas as pl

N = 10000
E = 160000
D = 256
OUT = 40
K = 4


def _cheb_norm(src, dst, n):
    w = (src != dst).astype(jnp.float32)
    deg = jax.ops.segment_sum(w, src, num_segments=n)
    dis = jnp.where(deg > 0, 1.0 / jnp.sqrt(jnp.maximum(deg, 1e-12)), 0.0)
    return -dis[src] * w * dis[dst]


def _lhat(x, src, dst, norm, n):
    return jax.ops.segment_sum(norm[:, None] * x[src], dst, num_segments=n)


def _cheb_conv(x, src, dst, norm, W, b, n):
    tx0 = x
    tx1 = _lhat(x, src, dst, norm, n)
    out = tx0 @ W[0] + tx1 @ W[1]
    for k in range(2, K):
        tx2 = 2.0 * _lhat(tx1, src, dst, norm, n) - tx0
        out = out + tx2 @ W[k]
        tx0, tx1 = tx1, tx2
    return out + b


def _graph_norm(x, g, b, ms, eps=1e-5):
    mean = x.mean(axis=0, keepdims=True)
    out = x - mean * ms
    var = (out * out).mean(axis=0, keepdims=True)
    return out / jnp.sqrt(var + eps) * g + b


def _head_body(h_ref, w_ref, b_ref, o_ref):
    h = jnp.maximum(h_ref[...], 0.0)
    z = jax.lax.dot_general(h, w_ref[...], (((1,), (0,)), ((), ()))) + b_ref[...]
    z = z - jnp.max(z)
    e = jnp.exp(z)
    o_ref[...] = e / jnp.sum(e)


def _head(hvec, lw, lb):
    out = pl.pallas_call(
        _head_body,
        out_shape=jax.ShapeDtypeStruct((1, OUT), jnp.float32),
    )(hvec[None, :], lw, lb[None, :])
    return out[0]


def _branch(feat, src, dst, norm, W, b, gg, gb, gms, lw, lb, n):
    h = feat
    for i in range(3):
        h = _cheb_conv(h, src, dst, norm, W[i], b[i], n)
        h = _graph_norm(h, gg[i], gb[i], gms[i])
        h = jax.nn.leaky_relu(h, 0.1)
    h = _cheb_conv(h, src, dst, norm, W[3], b[3], n)
    h = feat + _graph_norm(h, gg[3], gb[3], gms[3])
    h = jax.nn.relu(h)
    h = h.mean(axis=0)
    return _head(h, lw, lb)


def kernel(edge_index, feat, feat_, W1, b1, gn1_g, gn1_b, gn1_ms, lin1_w, lin1_b,
           W2, b2, gn2_g, gn2_b, gn2_ms, lin2_w, lin2_b):
    src, dst = edge_index[0], edge_index[1]
    n = feat.shape[0]
    norm = _cheb_norm(src, dst, n)
    out1 = _branch(feat, src, dst, norm, W1, b1, gn1_g, gn1_b, gn1_ms, lin1_w, lin1_b, n)
    out2 = _branch(feat_, src, dst, norm, W2, b2, gn2_g, gn2_b, gn2_ms, lin2_w, lin2_b, n)
    return (out1, out2)


# speedup vs baseline: 4.7060x; 4.7060x over previous
import functools

import jax
import jax.numpy as jnp
from jax import lax
from jax.experimental import pallas as pl
from jax.experimental.pallas import tpu as pltpu
from jax.experimental.pallas import tpu_sc as plsc

N = 10000
E = 160000
D = 256
OUT = 40
NPAD = 10240
ACCN = 10112
TRASH = N
NB = 250
BB = 40
NBUF = 3
SBB = 25
NSB = 10
DB = 40
DNB = 125
MB = 400
NMB = 25
EPS = 1e-5

_MESH = plsc.VectorSubcoreMesh(
    core_axis_name="c", subcore_axis_name="s", num_cores=2, num_subcores=16)


def _prop_body(xs_hbm, src_hbm, dst_hbm, out_hbm,
               src_v, dst_v, rows_v, acc,
               gsem0, gsem1, ssem0, ssem1, isem):
    c = lax.axis_index("c")
    s = lax.axis_index("s")

    for qi in range(2):
        q = 2 * c + qi

        def _gath(j, bi, sem):
            p = (j // SBB) % 2
            r = j % SBB
            return pltpu.async_copy(
                xs_hbm.at[src_v.at[p, r]], rows_v.at[bi], sem)

        def _gath_wait(j, bi, sem):
            p = (j // SBB) % 2
            r = j % SBB
            pltpu.make_async_copy(
                xs_hbm.at[src_v.at[p, r]], rows_v.at[bi], sem).wait()

        def _scat(j, bi, sem):
            p = (j // SBB) % 2
            r = j % SBB
            return pltpu.async_copy(
                rows_v.at[bi], acc.at[dst_v.at[p, r]], sem, add=True)

        def _scat_wait(bi, sem):
            pltpu.make_async_copy(
                rows_v.at[bi], acc.at[dst_v.at[0, 0]], sem).wait()

        def _idx_rows(sb, p, start):
            sbase = q * E + s * (E // 16) + sb * (SBB * BB)
            dbase = s * (E // 16) + sb * (SBB * BB)
            cps = []
            for r in range(SBB):
                cps.append(start(src_hbm.at[pl.ds(sbase + r * BB, BB)],
                                 src_v.at[p, r]))
                cps.append(start(dst_hbm.at[pl.ds(dbase + r * BB, BB)],
                                 dst_v.at[p, r]))
            return cps

        def _idx_wait(j):
            sb = j // SBB
            r = j % SBB

            @pl.when((r == 0) & (sb > 0) & (sb < NSB))
            def _wi():
                p = sb % 2
                for cp in _idx_rows(
                        sb, p, lambda a, b: pltpu.make_async_copy(a, b, isem)):
                    cp.wait()

        def _idx_prefetch(j):
            sb = j // SBB
            r = j % SBB

            @pl.when((r == 2) & (sb + 1 < NSB))
            def _pi():
                pn = (sb + 1) % 2
                _idx_rows(sb + 1, pn,
                          lambda a, b: pltpu.async_copy(a, b, isem))

        def _z(i, _):
            for l in range(8):
                rows_v[0, i, pl.ds(l * 16, 16)] = jnp.zeros((16,), jnp.float32)
            return _
        lax.fori_loop(0, BB, _z, None)
        for j in range(16):
            r = BB if j < 15 else 32
            pltpu.sync_copy(rows_v.at[0].at[pl.ds(0, r)],
                            acc.at[pl.ds(s * 632 + j * BB, r)])

        for r in range(SBB):
            pltpu.sync_copy(
                src_hbm.at[pl.ds(q * E + s * (E // 16) + r * BB, BB)],
                src_v.at[0, r])
            pltpu.sync_copy(
                dst_hbm.at[pl.ds(s * (E // 16) + r * BB, BB)],
                dst_v.at[0, r])
        plsc.subcore_barrier()

        _gath(0, 0, gsem0)

        def _pair(g, _):
            j0 = 2 * g
            j1 = j0 + 1
            j2 = j0 + 2

            @pl.when(g > 0)
            def _w1():
                _scat_wait(1, ssem1)
            _idx_wait(j1)
            _idx_prefetch(j1)
            _gath(j1, 1, gsem1)
            _gath_wait(j0, 0, gsem0)
            _scat(j0, 0, ssem0)
            _idx_wait(j2)
            _idx_prefetch(j2)
            _scat_wait(0, ssem0)

            @pl.when(j2 < NB)
            def _g2():
                _gath(j2, 0, gsem0)
            _gath_wait(j1, 1, gsem1)
            _scat(j1, 1, ssem1)
            return _
        lax.fori_loop(0, NB // 2, _pair, None)
        _scat_wait(1, ssem1)
        plsc.subcore_barrier()

        for j in range(16):
            r = BB if j < 15 else 32
            base = s * 632 + j * BB
            pltpu.sync_copy(acc.at[pl.ds(base, r)],
                            rows_v.at[0].at[pl.ds(0, r)])
            pltpu.sync_copy(rows_v.at[0].at[pl.ds(0, r)],
                            out_hbm.at[q].at[pl.ds(base, r)])
        plsc.subcore_barrier()


_prop_kernel = pl.kernel(
    _prop_body,
    out_type=jax.ShapeDtypeStruct((4, ACCN, 128), jnp.float32),
    mesh=_MESH,
    scratch_types=[
        pltpu.VMEM((2, SBB, BB), jnp.int32),
        pltpu.VMEM((2, SBB, BB), jnp.int32),
        pltpu.VMEM((2, BB, 128), jnp.float32),
        pltpu.VMEM_SHARED((ACCN, 128), jnp.float32),
        pltpu.SemaphoreType.DMA,
        pltpu.SemaphoreType.DMA,
        pltpu.SemaphoreType.DMA,
        pltpu.SemaphoreType.DMA,
        pltpu.SemaphoreType.DMA,
    ],
)


def _prep_edges_body(src_ref, dst_ref, deff_ref, seff_ref, srcq_ref):
    qq = pl.program_id(0)
    srcv = src_ref[...]
    dstv = dst_ref[...]
    loop = srcv == dstv
    deff_ref[...] = jnp.where(loop, TRASH, dstv)
    seff_ref[...] = jnp.where(loop, TRASH, srcv)
    srcq_ref[0] = srcv + qq * NPAD


def _prep_edges(src2, dst2):
    return pl.pallas_call(
        _prep_edges_body,
        grid=(4,),
        in_specs=[
            pl.BlockSpec((1250, 128), lambda qq: (0, 0)),
            pl.BlockSpec((1250, 128), lambda qq: (0, 0)),
        ],
        out_specs=(
            pl.BlockSpec((1250, 128), lambda qq: (0, 0)),
            pl.BlockSpec((1250, 128), lambda qq: (0, 0)),
            pl.BlockSpec((1, 1250, 128), lambda qq: (qq, 0, 0)),
        ),
        out_shape=(jax.ShapeDtypeStruct((1250, 128), jnp.int32),
                   jax.ShapeDtypeStruct((1250, 128), jnp.int32),
                   jax.ShapeDtypeStruct((4, 1250, 128), jnp.int32)),
    )(src2, dst2)


def _prep_nodes_body(degp_ref, f1_ref, f2_ref, c0_ref, u0_ref, dis_ref):
    deg = degp_ref[:, 0:1]
    dis = jnp.where(deg > 0, lax.rsqrt(jnp.maximum(deg, 1e-12)), 0.0)
    dis_ref[...] = dis
    f1 = f1_ref[...]
    f2 = f2_ref[...]
    planes = jnp.stack([f1[:, :128], f1[:, 128:], f2[:, :128], f2[:, 128:]])
    c0_ref[...] = planes
    u0_ref[...] = dis[None] * planes


def _prep_nodes(degp, feat, feat_):
    return pl.pallas_call(
        _prep_nodes_body,
        grid=(NMB,),
        in_specs=[
            pl.BlockSpec((MB, 128), lambda m: (m, 0)),
            pl.BlockSpec((MB, D), lambda m: (m, 0)),
            pl.BlockSpec((MB, D), lambda m: (m, 0)),
        ],
        out_specs=(
            pl.BlockSpec((4, MB, 128), lambda m: (0, m, 0)),
            pl.BlockSpec((4, MB, 128), lambda m: (0, m, 0)),
            pl.BlockSpec((MB, 1), lambda m: (m, 0)),
        ),
        out_shape=(jax.ShapeDtypeStruct((4, NPAD, 128), jnp.float32),
                   jax.ShapeDtypeStruct((4, NPAD, 128), jnp.float32),
                   jax.ShapeDtypeStruct((NPAD, 1), jnp.float32)),
    )(degp, feat, feat_)


def _stage1_body(r1_ref, dis_ref, o_ref):
    dis = dis_ref[...]
    o_ref[...] = (-dis * dis)[None] * r1_ref[...]


def _stage1(r1, dis2):
    return pl.pallas_call(
        _stage1_body,
        grid=(4, NMB),
        in_specs=[
            pl.BlockSpec((1, MB, 128), lambda q, m: (q, m, 0)),
            pl.BlockSpec((MB, 1), lambda q, m: (m, 0)),
        ],
        out_specs=pl.BlockSpec((1, MB, 128), lambda q, m: (q, m, 0)),
        out_shape=jax.ShapeDtypeStruct((4, NPAD, 128), jnp.float32),
    )(r1.reshape(4, ACCN, 128), dis2)


def _stage2_body(r2_ref, u0_ref, dis_ref, o_ref):
    dis = dis_ref[...]
    o_ref[...] = (-2.0 * dis * dis)[None] * r2_ref[...] - u0_ref[...]


def _stage2(r2, u0, dis2):
    return pl.pallas_call(
        _stage2_body,
        grid=(4, NMB),
        in_specs=[
            pl.BlockSpec((1, MB, 128), lambda q, m: (q, m, 0)),
            pl.BlockSpec((1, MB, 128), lambda q, m: (q, m, 0)),
            pl.BlockSpec((MB, 1), lambda q, m: (m, 0)),
        ],
        out_specs=pl.BlockSpec((1, MB, 128), lambda q, m: (q, m, 0)),
        out_shape=jax.ShapeDtypeStruct((4, NPAD, 128), jnp.float32),
    )(r2.reshape(4, ACCN, 128), u0, dis2)


def _matmul_body(c0_ref, r1_ref, r2_ref, r3_ref, dis_ref, w_ref, b_ref, o_ref):
    dis = dis_ref[...]
    cat = lambda a: jnp.concatenate([a[0], a[1]], axis=1)
    tx0 = cat(c0_ref[...])
    r1 = cat(r1_ref[...])
    tx1 = -dis * r1
    tx2 = -2.0 * dis * cat(r2_ref[...]) - tx0
    tx3 = -2.0 * dis * cat(r3_ref[...]) - tx1
    w = w_ref[0]
    acc = jnp.dot(tx0, w[0], preferred_element_type=jnp.float32)
    acc += jnp.dot(tx1, w[1], preferred_element_type=jnp.float32)
    acc += jnp.dot(tx2, w[2], preferred_element_type=jnp.float32)
    acc += jnp.dot(tx3, w[3], preferred_element_type=jnp.float32)
    o_ref[0] = acc + b_ref[0]


def _cheb_matmul(c0, r1, r2, r3, dis2, wl, bl):
    pspec = pl.BlockSpec((2, MB, 128), lambda b, m: (b, m, 0))
    return pl.pallas_call(
        _matmul_body,
        grid=(2, NMB),
        in_specs=[
            pspec, pspec, pspec, pspec,
            pl.BlockSpec((MB, 1), lambda b, m: (m, 0)),
            pl.BlockSpec((1, 4, D, D), lambda b, m: (b, 0, 0, 0)),
            pl.BlockSpec((1, 1, D), lambda b, m: (b, 0, 0)),
        ],
        out_specs=pl.BlockSpec((1, MB, D), lambda b, m: (b, m, 0)),
        out_shape=jax.ShapeDtypeStruct((2, N, D), jnp.float32),
    )(c0, r1.reshape(4, ACCN, 128), r2.reshape(4, ACCN, 128),
      r3.reshape(4, ACCN, 128), dis2, wl, bl)


def _stats_body(conv_ref, gms_ref, o_ref):
    m = pl.program_id(1)
    x = conv_ref[0]
    s1 = jnp.sum(x, axis=0, keepdims=True)
    s2 = jnp.sum(x * x, axis=0, keepdims=True)
    blk = jnp.concatenate([s1, s2, jnp.zeros((6, D), jnp.float32)], 0)[None]

    @pl.when(m == 0)
    def _i():
        o_ref[...] = blk

    @pl.when(m > 0)
    def _a():
        o_ref[...] += blk

    @pl.when(m == NMB - 1)
    def _f():
        tot = o_ref[...]
        mean = tot[0, 0:1, :] / N
        ex2 = tot[0, 1:2, :] / N
        shift = mean * gms_ref[0]
        var = ex2 - 2.0 * shift * mean + shift * shift
        inv = lax.rsqrt(var + EPS)
        o_ref[...] = jnp.concatenate(
            [shift, inv, jnp.zeros((6, D), jnp.float32)], 0)[None]


def _stats(conv, gms):
    return pl.pallas_call(
        _stats_body,
        grid=(2, NMB),
        in_specs=[
            pl.BlockSpec((1, MB, D), lambda b, m: (b, m, 0)),
            pl.BlockSpec((1, 1, D), lambda b, m: (b, 0, 0)),
        ],
        out_specs=pl.BlockSpec((1, 8, D), lambda b, m: (b, 0, 0)),
        out_shape=jax.ShapeDtypeStruct((2, 8, D), jnp.float32),
    )(conv, gms)


def _norm_body(conv_ref, st_ref, g_ref, b_ref, dis_ref, c0_ref, u_ref):
    x = conv_ref[0]
    shift = st_ref[0, 0:1, :]
    inv = st_ref[0, 1:2, :]
    y = (x - shift) * inv * g_ref[0] + b_ref[0]
    h = jnp.where(y >= 0, y, 0.1 * y)
    c0_ref[0] = h
    u_ref[0] = dis_ref[...] * h


def _normalize(conv, st, gg, gb, dis2):
    return pl.pallas_call(
        _norm_body,
        grid=(4, NMB),
        in_specs=[
            pl.BlockSpec((1, MB, 128), lambda q, m: (q // 2, m, q % 2)),
            pl.BlockSpec((1, 8, 128), lambda q, m: (q // 2, 0, q % 2)),
            pl.BlockSpec((1, 1, 128), lambda q, m: (q // 2, 0, q % 2)),
            pl.BlockSpec((1, 1, 128), lambda q, m: (q // 2, 0, q % 2)),
            pl.BlockSpec((MB, 1), lambda q, m: (m, 0)),
        ],
        out_specs=(
            pl.BlockSpec((1, MB, 128), lambda q, m: (q, m, 0)),
            pl.BlockSpec((1, MB, 128), lambda q, m: (q, m, 0)),
        ),
        out_shape=(jax.ShapeDtypeStruct((4, NPAD, 128), jnp.float32),
                   jax.ShapeDtypeStruct((4, NPAD, 128), jnp.float32)),
    )(conv, st, gg, gb, dis2)


def _final_body(conv_ref, st_ref, g_ref, b_ref, feat_ref, o_ref):
    m = pl.program_id(1)
    x = conv_ref[0]
    shift = st_ref[0, 0:1, :]
    inv = st_ref[0, 1:2, :]
    y = feat_ref[0] + (x - shift) * inv * g_ref[0] + b_ref[0]
    h = jnp.maximum(y, 0.0)
    blk = jnp.concatenate(
        [jnp.sum(h, axis=0, keepdims=True), jnp.zeros((7, D), jnp.float32)],
        0)[None]

    @pl.when(m == 0)
    def _i():
        o_ref[...] = blk

    @pl.when(m > 0)
    def _a():
        o_ref[...] += blk


def _final_phase(conv, st, gg, gb, feats):
    return pl.pallas_call(
        _final_body,
        grid=(2, NMB),
        in_specs=[
            pl.BlockSpec((1, MB, D), lambda b, m: (b, m, 0)),
            pl.BlockSpec((1, 8, D), lambda b, m: (b, 0, 0)),
            pl.BlockSpec((1, 1, D), lambda b, m: (b, 0, 0)),
            pl.BlockSpec((1, 1, D), lambda b, m: (b, 0, 0)),
            pl.BlockSpec((1, MB, D), lambda b, m: (b, m, 0)),
        ],
        out_specs=pl.BlockSpec((1, 8, D), lambda b, m: (b, 0, 0)),
        out_shape=jax.ShapeDtypeStruct((2, 8, D), jnp.float32),
    )(conv, st, gg, gb, feats)


def _head_body(cs_ref, lw_ref, lb_ref, o_ref):
    hm = jnp.maximum(cs_ref[0, 0:1, :] / N, 0.0)
    z = jnp.dot(hm, lw_ref[0], preferred_element_type=jnp.float32)
    z = z + lb_ref[0]
    z = z - jnp.max(z, axis=-1, keepdims=True)
    e = jnp.exp(z)
    o_ref[0] = e / jnp.sum(e, axis=-1, keepdims=True)


def _head(colsum, lwc, lbc):
    return pl.pallas_call(
        _head_body,
        grid=(2,),
        in_specs=[
            pl.BlockSpec((1, 8, D), lambda b: (b, 0, 0)),
            pl.BlockSpec((1, D, OUT), lambda b: (b, 0, 0)),
            pl.BlockSpec((1, 1, OUT), lambda b: (b, 0, 0)),
        ],
        out_specs=pl.BlockSpec((1, 1, OUT), lambda b: (b, 0, 0)),
        out_shape=jax.ShapeDtypeStruct((2, 1, OUT), jnp.float32),
    )(colsum, lwc, lbc)


def kernel(edge_index, feat, feat_, W1, b1, gn1_g, gn1_b, gn1_ms, lin1_w, lin1_b,
           W2, b2, gn2_g, gn2_b, gn2_ms, lin2_w, lin2_b):
    src, dst = edge_index[0], edge_index[1]
    src2 = src.reshape(1250, 128)
    dst2 = dst.reshape(1250, 128)
    deff2, seff2, srcq = _prep_edges(src2, dst2)

    src4 = srcq.reshape(4 * E)
    dst4 = deff2.reshape(E)

    ones_tab = jnp.ones((4 * NPAD, 128), jnp.float32)
    degp = _prop_kernel(ones_tab, src4, seff2.reshape(E))[0]
    c0, u, dis2 = _prep_nodes(degp, feat, feat_)

    Wc = jnp.stack([W1, W2])
    bc = jnp.stack([b1, b2]).reshape(2, 4, 1, D)
    gc = jnp.stack([gn1_g, gn2_g]).reshape(2, 4, 1, D)
    bgc = jnp.stack([gn1_b, gn2_b]).reshape(2, 4, 1, D)
    msc = jnp.stack([gn1_ms, gn2_ms]).reshape(2, 4, 1, D)
    feats = jnp.stack([feat, feat_])
    lwc = jnp.stack([lin1_w, lin2_w])
    lbc = jnp.stack([lin1_b, lin2_b]).reshape(2, 1, OUT)

    for i in range(4):
        r1 = _prop_kernel(u.reshape(4 * NPAD, 128), src4, dst4)
        in2 = _stage1(r1, dis2)
        r2 = _prop_kernel(in2.reshape(4 * NPAD, 128), src4, dst4)
        in3 = _stage2(r2, u, dis2)
        r3 = _prop_kernel(in3.reshape(4 * NPAD, 128), src4, dst4)
        conv = _cheb_matmul(c0, r1, r2, r3, dis2, Wc[:, i], bc[:, i])
        st = _stats(conv, msc[:, i])
        if i < 3:
            c0, u = _normalize(conv, st, gc[:, i], bgc[:, i], dis2)
        else:
            colsum = _final_phase(conv, st, gc[:, i], bgc[:, i], feats)

    out = _head(colsum, lwc, lbc)
    return (out[0, 0], out[1, 0])


# speedup vs baseline: 4.9767x; 1.0575x over previous
import functools

import jax
import jax.numpy as jnp
from jax import lax
from jax.experimental import pallas as pl
from jax.experimental.pallas import tpu as pltpu
from jax.experimental.pallas import tpu_sc as plsc

N = 10000
E = 160000
D = 256
OUT = 40
NPAD = 10240
ACCN = 10112
TRASH = N
NB = 250
BB = 40
NBUF = 3
SBB = 25
NSB = 10
DB = 40
DNB = 125
MB = 400
NMB = 25
EPS = 1e-5

_MESH = plsc.VectorSubcoreMesh(
    core_axis_name="c", subcore_axis_name="s", num_cores=2, num_subcores=16)


def _prop_body(xs_hbm, src_hbm, dst_hbm, out_hbm,
               src_v, dst_v, rows_v, acc,
               gsem0, gsem1, ssem0, ssem1, isem):
    c = lax.axis_index("c")
    s = lax.axis_index("s")

    for qi in range(2):
        q = 2 * c + qi

        def _gath(j, bi, sem):
            p = (j // SBB) % 2
            r = j % SBB
            return pltpu.async_copy(
                xs_hbm.at[src_v.at[p, r]], rows_v.at[bi], sem)

        def _gath_wait(j, bi, sem):
            p = (j // SBB) % 2
            r = j % SBB
            pltpu.make_async_copy(
                xs_hbm.at[src_v.at[p, r]], rows_v.at[bi], sem).wait()

        def _scat(j, bi, sem):
            p = (j // SBB) % 2
            r = j % SBB
            return pltpu.async_copy(
                rows_v.at[bi], acc.at[dst_v.at[p, r]], sem, add=True)

        def _scat_wait(bi, sem):
            pltpu.make_async_copy(
                rows_v.at[bi], acc.at[dst_v.at[0, 0]], sem).wait()

        def _idx_rows(sb, p, start):
            sbase = q * E + s * (E // 16) + sb * (SBB * BB)
            dbase = s * (E // 16) + sb * (SBB * BB)
            cps = []
            for r in range(SBB):
                cps.append(start(src_hbm.at[pl.ds(sbase + r * BB, BB)],
                                 src_v.at[p, r]))
                cps.append(start(dst_hbm.at[pl.ds(dbase + r * BB, BB)],
                                 dst_v.at[p, r]))
            return cps

        def _idx_wait(j):
            sb = j // SBB
            r = j % SBB

            @pl.when((r == 0) & (sb > 0) & (sb < NSB))
            def _wi():
                p = sb % 2
                for cp in _idx_rows(
                        sb, p, lambda a, b: pltpu.make_async_copy(a, b, isem)):
                    cp.wait()

        def _idx_prefetch(j):
            sb = j // SBB
            r = j % SBB

            @pl.when((r == 2) & (sb + 1 < NSB))
            def _pi():
                pn = (sb + 1) % 2
                _idx_rows(sb + 1, pn,
                          lambda a, b: pltpu.async_copy(a, b, isem))

        def _z(i, _):
            for l in range(8):
                rows_v[0, i, pl.ds(l * 16, 16)] = jnp.zeros((16,), jnp.float32)
            return _
        lax.fori_loop(0, BB, _z, None)
        for j in range(16):
            r = BB if j < 15 else 32
            pltpu.sync_copy(rows_v.at[0].at[pl.ds(0, r)],
                            acc.at[pl.ds(s * 632 + j * BB, r)])

        for r in range(SBB):
            pltpu.sync_copy(
                src_hbm.at[pl.ds(q * E + s * (E // 16) + r * BB, BB)],
                src_v.at[0, r])
            pltpu.sync_copy(
                dst_hbm.at[pl.ds(s * (E // 16) + r * BB, BB)],
                dst_v.at[0, r])
        plsc.subcore_barrier()

        _gath(0, 0, gsem0)

        def _pair(g, _):
            j0 = 2 * g
            j1 = j0 + 1
            j2 = j0 + 2

            @pl.when(g > 0)
            def _w1():
                _scat_wait(1, ssem1)
            _idx_wait(j1)
            _idx_prefetch(j1)
            _gath(j1, 1, gsem1)
            _gath_wait(j0, 0, gsem0)
            _scat(j0, 0, ssem0)
            _idx_wait(j2)
            _idx_prefetch(j2)
            _scat_wait(0, ssem0)

            @pl.when(j2 < NB)
            def _g2():
                _gath(j2, 0, gsem0)
            _gath_wait(j1, 1, gsem1)
            _scat(j1, 1, ssem1)
            return _
        lax.fori_loop(0, NB // 2, _pair, None)
        _scat_wait(1, ssem1)
        plsc.subcore_barrier()

        for j in range(16):
            r = BB if j < 15 else 32
            base = s * 632 + j * BB
            pltpu.sync_copy(acc.at[pl.ds(base, r)],
                            rows_v.at[0].at[pl.ds(0, r)])
            pltpu.sync_copy(rows_v.at[0].at[pl.ds(0, r)],
                            out_hbm.at[q].at[pl.ds(base, r)])
        plsc.subcore_barrier()


_prop_kernel = pl.kernel(
    _prop_body,
    out_type=jax.ShapeDtypeStruct((4, ACCN, 128), jnp.float32),
    mesh=_MESH,
    scratch_types=[
        pltpu.VMEM((2, SBB, BB), jnp.int32),
        pltpu.VMEM((2, SBB, BB), jnp.int32),
        pltpu.VMEM((2, BB, 128), jnp.float32),
        pltpu.VMEM_SHARED((ACCN, 128), jnp.float32),
        pltpu.SemaphoreType.DMA,
        pltpu.SemaphoreType.DMA,
        pltpu.SemaphoreType.DMA,
        pltpu.SemaphoreType.DMA,
        pltpu.SemaphoreType.DMA,
    ],
)


def _deg_body(dst_hbm, out_hbm, dst_v, ones_v, zbuf, acc, ssem0, ssem1, isem):
    c = lax.axis_index("c")
    s = lax.axis_index("s")
    EPC = E // 2
    DPT = EPC // 16
    DNB = DPT // BB
    DNSB = DNB // SBB

    def _fill(ref, val):
        def _f(i, _):
            for l in range(8):
                ref[i, pl.ds(l * 16, 16)] = jnp.full((16,), val, jnp.float32)
            return _
        lax.fori_loop(0, BB, _f, None)

    _fill(ones_v, 1.0)
    _fill(zbuf, 0.0)
    for j in range(16):
        r = BB if j < 15 else 32
        pltpu.sync_copy(zbuf.at[pl.ds(0, r)],
                        acc.at[pl.ds(s * 632 + j * BB, r)])

    def _scat(j, sem):
        p = (j // SBB) % 2
        r = j % SBB
        return pltpu.async_copy(
            ones_v, acc.at[dst_v.at[p, r]], sem, add=True)

    def _scat_wait(sem):
        pltpu.make_async_copy(ones_v, acc.at[dst_v.at[0, 0]], sem).wait()

    def _idx_rows(sb, p, start):
        dbase = c * EPC + s * DPT + sb * (SBB * BB)
        return [start(dst_hbm.at[pl.ds(dbase + r * BB, BB)], dst_v.at[p, r])
                for r in range(SBB)]

    def _idx_wait(j):
        sb = j // SBB
        r = j % SBB

        @pl.when((r == 0) & (sb > 0) & (sb < DNSB))
        def _wi():
            for cp in _idx_rows(
                    sb, sb % 2,
                    lambda a, b: pltpu.make_async_copy(a, b, isem)):
                cp.wait()

    def _idx_prefetch(j):
        sb = j // SBB
        r = j % SBB

        @pl.when((r == 4) & (sb + 1 < DNSB))
        def _pi():
            _idx_rows(sb + 1, (sb + 1) % 2,
                      lambda a, b: pltpu.async_copy(a, b, isem))

    for r in range(SBB):
        pltpu.sync_copy(dst_hbm.at[pl.ds(c * EPC + s * DPT + r * BB, BB)],
                        dst_v.at[0, r])
    plsc.subcore_barrier()

    def _pair(g, _):
        j0 = 2 * g
        j1 = j0 + 1

        @pl.when(g > 1)
        def _w0():
            _scat_wait(ssem0)
        _idx_wait(j0)
        _idx_prefetch(j0)
        _scat(j0, ssem0)

        @pl.when(g > 1)
        def _w1():
            _scat_wait(ssem1)
        _idx_wait(j1)
        _idx_prefetch(j1)
        _scat(j1, ssem1)
        return _
    lax.fori_loop(0, DNB // 2, _pair, None)
    _idx_wait(DNB - 1)
    _scat(DNB - 1, ssem0)
    for _ in range(3):
        _scat_wait(ssem0)
    for _ in range(2):
        _scat_wait(ssem1)
    plsc.subcore_barrier()

    for j in range(16):
        r = BB if j < 15 else 32
        base = s * 632 + j * BB
        pltpu.sync_copy(acc.at[pl.ds(base, r)], zbuf.at[pl.ds(0, r)])
        pltpu.sync_copy(zbuf.at[pl.ds(0, r)],
                        out_hbm.at[c].at[pl.ds(base, r)])


_deg_kernel = pl.kernel(
    _deg_body,
    out_type=jax.ShapeDtypeStruct((2, ACCN, 128), jnp.float32),
    mesh=_MESH,
    scratch_types=[
        pltpu.VMEM((2, SBB, BB), jnp.int32),
        pltpu.VMEM((BB, 128), jnp.float32),
        pltpu.VMEM((BB, 128), jnp.float32),
        pltpu.VMEM_SHARED((ACCN, 128), jnp.float32),
        pltpu.SemaphoreType.DMA,
        pltpu.SemaphoreType.DMA,
        pltpu.SemaphoreType.DMA,
    ],
)


def _prep_edges_body(src_ref, dst_ref, deff_ref, seff_ref, srcq_ref):
    qq = pl.program_id(0)
    srcv = src_ref[...]
    dstv = dst_ref[...]
    loop = srcv == dstv
    deff_ref[...] = jnp.where(loop, TRASH, dstv)
    seff_ref[...] = jnp.where(loop, TRASH, srcv)
    srcq_ref[0] = srcv + qq * NPAD


def _prep_edges(src2, dst2):
    return pl.pallas_call(
        _prep_edges_body,
        grid=(4,),
        in_specs=[
            pl.BlockSpec((1250, 128), lambda qq: (0, 0)),
            pl.BlockSpec((1250, 128), lambda qq: (0, 0)),
        ],
        out_specs=(
            pl.BlockSpec((1250, 128), lambda qq: (0, 0)),
            pl.BlockSpec((1250, 128), lambda qq: (0, 0)),
            pl.BlockSpec((1, 1250, 128), lambda qq: (qq, 0, 0)),
        ),
        out_shape=(jax.ShapeDtypeStruct((1250, 128), jnp.int32),
                   jax.ShapeDtypeStruct((1250, 128), jnp.int32),
                   jax.ShapeDtypeStruct((4, 1250, 128), jnp.int32)),
    )(src2, dst2)


def _prep_nodes_body(degp_ref, f1_ref, f2_ref, c0_ref, u0_ref, dis_ref):
    deg = degp_ref[0, :, 0:1] + degp_ref[1, :, 0:1]
    dis = jnp.where(deg > 0, lax.rsqrt(jnp.maximum(deg, 1e-12)), 0.0)
    dis_ref[...] = dis
    f1 = f1_ref[...]
    f2 = f2_ref[...]
    planes = jnp.stack([f1[:, :128], f1[:, 128:], f2[:, :128], f2[:, 128:]])
    c0_ref[...] = planes
    u0_ref[...] = dis[None] * planes


def _prep_nodes(degp, feat, feat_):
    return pl.pallas_call(
        _prep_nodes_body,
        grid=(NMB,),
        in_specs=[
            pl.BlockSpec((2, MB, 128), lambda m: (0, m, 0)),
            pl.BlockSpec((MB, D), lambda m: (m, 0)),
            pl.BlockSpec((MB, D), lambda m: (m, 0)),
        ],
        out_specs=(
            pl.BlockSpec((4, MB, 128), lambda m: (0, m, 0)),
            pl.BlockSpec((4, MB, 128), lambda m: (0, m, 0)),
            pl.BlockSpec((MB, 1), lambda m: (m, 0)),
        ),
        out_shape=(jax.ShapeDtypeStruct((4, NPAD, 128), jnp.float32),
                   jax.ShapeDtypeStruct((4, NPAD, 128), jnp.float32),
                   jax.ShapeDtypeStruct((NPAD, 1), jnp.float32)),
    )(degp, feat, feat_)


def _stage1_body(r1_ref, dis_ref, o_ref):
    dis = dis_ref[...]
    o_ref[...] = (-dis * dis)[None] * r1_ref[...]


def _stage1(r1, dis2):
    return pl.pallas_call(
        _stage1_body,
        grid=(4, NMB),
        in_specs=[
            pl.BlockSpec((1, MB, 128), lambda q, m: (q, m, 0)),
            pl.BlockSpec((MB, 1), lambda q, m: (m, 0)),
        ],
        out_specs=pl.BlockSpec((1, MB, 128), lambda q, m: (q, m, 0)),
        out_shape=jax.ShapeDtypeStruct((4, NPAD, 128), jnp.float32),
    )(r1.reshape(4, ACCN, 128), dis2)


def _stage2_body(r2_ref, u0_ref, dis_ref, o_ref):
    dis = dis_ref[...]
    o_ref[...] = (-2.0 * dis * dis)[None] * r2_ref[...] - u0_ref[...]


def _stage2(r2, u0, dis2):
    return pl.pallas_call(
        _stage2_body,
        grid=(4, NMB),
        in_specs=[
            pl.BlockSpec((1, MB, 128), lambda q, m: (q, m, 0)),
            pl.BlockSpec((1, MB, 128), lambda q, m: (q, m, 0)),
            pl.BlockSpec((MB, 1), lambda q, m: (m, 0)),
        ],
        out_specs=pl.BlockSpec((1, MB, 128), lambda q, m: (q, m, 0)),
        out_shape=jax.ShapeDtypeStruct((4, NPAD, 128), jnp.float32),
    )(r2.reshape(4, ACCN, 128), u0, dis2)


def _matmul_body(c0_ref, r1_ref, r2_ref, r3_ref, dis_ref, w_ref, b_ref, o_ref):
    dis = dis_ref[...]
    cat = lambda a: jnp.concatenate([a[0], a[1]], axis=1)
    tx0 = cat(c0_ref[...])
    r1 = cat(r1_ref[...])
    tx1 = -dis * r1
    tx2 = -2.0 * dis * cat(r2_ref[...]) - tx0
    tx3 = -2.0 * dis * cat(r3_ref[...]) - tx1
    w = w_ref[0]
    acc = jnp.dot(tx0, w[0], preferred_element_type=jnp.float32)
    acc += jnp.dot(tx1, w[1], preferred_element_type=jnp.float32)
    acc += jnp.dot(tx2, w[2], preferred_element_type=jnp.float32)
    acc += jnp.dot(tx3, w[3], preferred_element_type=jnp.float32)
    o_ref[0] = acc + b_ref[0]


def _cheb_matmul(c0, r1, r2, r3, dis2, wl, bl):
    pspec = pl.BlockSpec((2, MB, 128), lambda b, m: (b, m, 0))
    return pl.pallas_call(
        _matmul_body,
        grid=(2, NMB),
        in_specs=[
            pspec, pspec, pspec, pspec,
            pl.BlockSpec((MB, 1), lambda b, m: (m, 0)),
            pl.BlockSpec((1, 4, D, D), lambda b, m: (b, 0, 0, 0)),
            pl.BlockSpec((1, 1, D), lambda b, m: (b, 0, 0)),
        ],
        out_specs=pl.BlockSpec((1, MB, D), lambda b, m: (b, m, 0)),
        out_shape=jax.ShapeDtypeStruct((2, N, D), jnp.float32),
    )(c0, r1.reshape(4, ACCN, 128), r2.reshape(4, ACCN, 128),
      r3.reshape(4, ACCN, 128), dis2, wl, bl)


def _stats_body(conv_ref, gms_ref, o_ref):
    m = pl.program_id(1)
    x = conv_ref[0]
    s1 = jnp.sum(x, axis=0, keepdims=True)
    s2 = jnp.sum(x * x, axis=0, keepdims=True)
    blk = jnp.concatenate([s1, s2, jnp.zeros((6, D), jnp.float32)], 0)[None]

    @pl.when(m == 0)
    def _i():
        o_ref[...] = blk

    @pl.when(m > 0)
    def _a():
        o_ref[...] += blk

    @pl.when(m == NMB - 1)
    def _f():
        tot = o_ref[...]
        mean = tot[0, 0:1, :] / N
        ex2 = tot[0, 1:2, :] / N
        shift = mean * gms_ref[0]
        var = ex2 - 2.0 * shift * mean + shift * shift
        inv = lax.rsqrt(var + EPS)
        o_ref[...] = jnp.concatenate(
            [shift, inv, jnp.zeros((6, D), jnp.float32)], 0)[None]


def _stats(conv, gms):
    return pl.pallas_call(
        _stats_body,
        grid=(2, NMB),
        in_specs=[
            pl.BlockSpec((1, MB, D), lambda b, m: (b, m, 0)),
            pl.BlockSpec((1, 1, D), lambda b, m: (b, 0, 0)),
        ],
        out_specs=pl.BlockSpec((1, 8, D), lambda b, m: (b, 0, 0)),
        out_shape=jax.ShapeDtypeStruct((2, 8, D), jnp.float32),
    )(conv, gms)


def _norm_body(conv_ref, st_ref, g_ref, b_ref, dis_ref, c0_ref, u_ref):
    x = conv_ref[0]
    shift = st_ref[0, 0:1, :]
    inv = st_ref[0, 1:2, :]
    y = (x - shift) * inv * g_ref[0] + b_ref[0]
    h = jnp.where(y >= 0, y, 0.1 * y)
    c0_ref[0] = h
    u_ref[0] = dis_ref[...] * h


def _normalize(conv, st, gg, gb, dis2):
    return pl.pallas_call(
        _norm_body,
        grid=(4, NMB),
        in_specs=[
            pl.BlockSpec((1, MB, 128), lambda q, m: (q // 2, m, q % 2)),
            pl.BlockSpec((1, 8, 128), lambda q, m: (q // 2, 0, q % 2)),
            pl.BlockSpec((1, 1, 128), lambda q, m: (q // 2, 0, q % 2)),
            pl.BlockSpec((1, 1, 128), lambda q, m: (q // 2, 0, q % 2)),
            pl.BlockSpec((MB, 1), lambda q, m: (m, 0)),
        ],
        out_specs=(
            pl.BlockSpec((1, MB, 128), lambda q, m: (q, m, 0)),
            pl.BlockSpec((1, MB, 128), lambda q, m: (q, m, 0)),
        ),
        out_shape=(jax.ShapeDtypeStruct((4, NPAD, 128), jnp.float32),
                   jax.ShapeDtypeStruct((4, NPAD, 128), jnp.float32)),
    )(conv, st, gg, gb, dis2)


def _final_body(conv_ref, st_ref, g_ref, b_ref, feat_ref, o_ref):
    m = pl.program_id(1)
    x = conv_ref[0]
    shift = st_ref[0, 0:1, :]
    inv = st_ref[0, 1:2, :]
    y = feat_ref[0] + (x - shift) * inv * g_ref[0] + b_ref[0]
    h = jnp.maximum(y, 0.0)
    blk = jnp.concatenate(
        [jnp.sum(h, axis=0, keepdims=True), jnp.zeros((7, D), jnp.float32)],
        0)[None]

    @pl.when(m == 0)
    def _i():
        o_ref[...] = blk

    @pl.when(m > 0)
    def _a():
        o_ref[...] += blk


def _final_phase(conv, st, gg, gb, feats):
    return pl.pallas_call(
        _final_body,
        grid=(2, NMB),
        in_specs=[
            pl.BlockSpec((1, MB, D), lambda b, m: (b, m, 0)),
            pl.BlockSpec((1, 8, D), lambda b, m: (b, 0, 0)),
            pl.BlockSpec((1, 1, D), lambda b, m: (b, 0, 0)),
            pl.BlockSpec((1, 1, D), lambda b, m: (b, 0, 0)),
            pl.BlockSpec((1, MB, D), lambda b, m: (b, m, 0)),
        ],
        out_specs=pl.BlockSpec((1, 8, D), lambda b, m: (b, 0, 0)),
        out_shape=jax.ShapeDtypeStruct((2, 8, D), jnp.float32),
    )(conv, st, gg, gb, feats)


def _head_body(cs_ref, lw_ref, lb_ref, o_ref):
    hm = jnp.maximum(cs_ref[0, 0:1, :] / N, 0.0)
    z = jnp.dot(hm, lw_ref[0], preferred_element_type=jnp.float32)
    z = z + lb_ref[0]
    z = z - jnp.max(z, axis=-1, keepdims=True)
    e = jnp.exp(z)
    o_ref[0] = e / jnp.sum(e, axis=-1, keepdims=True)


def _head(colsum, lwc, lbc):
    return pl.pallas_call(
        _head_body,
        grid=(2,),
        in_specs=[
            pl.BlockSpec((1, 8, D), lambda b: (b, 0, 0)),
            pl.BlockSpec((1, D, OUT), lambda b: (b, 0, 0)),
            pl.BlockSpec((1, 1, OUT), lambda b: (b, 0, 0)),
        ],
        out_specs=pl.BlockSpec((1, 1, OUT), lambda b: (b, 0, 0)),
        out_shape=jax.ShapeDtypeStruct((2, 1, OUT), jnp.float32),
    )(colsum, lwc, lbc)


def kernel(edge_index, feat, feat_, W1, b1, gn1_g, gn1_b, gn1_ms, lin1_w, lin1_b,
           W2, b2, gn2_g, gn2_b, gn2_ms, lin2_w, lin2_b):
    src, dst = edge_index[0], edge_index[1]
    src2 = src.reshape(1250, 128)
    dst2 = dst.reshape(1250, 128)
    deff2, seff2, srcq = _prep_edges(src2, dst2)

    src4 = srcq.reshape(4 * E)
    dst4 = deff2.reshape(E)

    degp = _deg_kernel(seff2.reshape(E))
    c0, u, dis2 = _prep_nodes(degp, feat, feat_)

    Wc = jnp.stack([W1, W2])
    bc = jnp.stack([b1, b2]).reshape(2, 4, 1, D)
    gc = jnp.stack([gn1_g, gn2_g]).reshape(2, 4, 1, D)
    bgc = jnp.stack([gn1_b, gn2_b]).reshape(2, 4, 1, D)
    msc = jnp.stack([gn1_ms, gn2_ms]).reshape(2, 4, 1, D)
    feats = jnp.stack([feat, feat_])
    lwc = jnp.stack([lin1_w, lin2_w])
    lbc = jnp.stack([lin1_b, lin2_b]).reshape(2, 1, OUT)

    for i in range(4):
        r1 = _prop_kernel(u.reshape(4 * NPAD, 128), src4, dst4)
        in2 = _stage1(r1, dis2)
        r2 = _prop_kernel(in2.reshape(4 * NPAD, 128), src4, dst4)
        in3 = _stage2(r2, u, dis2)
        r3 = _prop_kernel(in3.reshape(4 * NPAD, 128), src4, dst4)
        conv = _cheb_matmul(c0, r1, r2, r3, dis2, Wc[:, i], bc[:, i])
        st = _stats(conv, msc[:, i])
        if i < 3:
            c0, u = _normalize(conv, st, gc[:, i], bgc[:, i], dis2)
        else:
            colsum = _final_phase(conv, st, gc[:, i], bgc[:, i], feats)

    out = _head(colsum, lwc, lbc)
    return (out[0, 0], out[1, 0])


# speedup vs baseline: 5.0649x; 1.0177x over previous
import functools

import jax
import jax.numpy as jnp
from jax import lax
from jax.experimental import pallas as pl
from jax.experimental.pallas import tpu as pltpu
from jax.experimental.pallas import tpu_sc as plsc

N = 10000
E = 160000
D = 256
OUT = 40
NPAD = 10240
ACCN = 10112
TRASH = N
NB = 250
BB = 40
NBUF = 3
SBB = 25
NSB = 10
DB = 40
DNB = 125
MB = 400
NMB = 25
EPS = 1e-5

_MESH = plsc.VectorSubcoreMesh(
    core_axis_name="c", subcore_axis_name="s", num_cores=2, num_subcores=16)


def _prop_body(xs_hbm, src_hbm, dst_hbm, out_hbm,
               src_v, dst_v, rows_v, acc,
               gsem0, gsem1, ssem0, ssem1, isem):
    c = lax.axis_index("c")
    s = lax.axis_index("s")

    for qi in range(2):
        q = 2 * c + qi

        def _gath(j, bi, sem):
            p = (j // SBB) % 2
            r = j % SBB
            return pltpu.async_copy(
                xs_hbm.at[src_v.at[p, r]], rows_v.at[bi], sem)

        def _gath_wait(j, bi, sem):
            p = (j // SBB) % 2
            r = j % SBB
            pltpu.make_async_copy(
                xs_hbm.at[src_v.at[p, r]], rows_v.at[bi], sem).wait()

        def _scat(j, bi, sem):
            p = (j // SBB) % 2
            r = j % SBB
            return pltpu.async_copy(
                rows_v.at[bi], acc.at[dst_v.at[p, r]], sem, add=True)

        def _scat_wait(bi, sem):
            pltpu.make_async_copy(
                rows_v.at[bi], acc.at[dst_v.at[0, 0]], sem).wait()

        def _idx_rows(sb, p, start):
            sbase = q * E + s * (E // 16) + sb * (SBB * BB)
            dbase = s * (E // 16) + sb * (SBB * BB)
            cps = []
            for r in range(SBB):
                cps.append(start(src_hbm.at[pl.ds(sbase + r * BB, BB)],
                                 src_v.at[p, r]))
                cps.append(start(dst_hbm.at[pl.ds(dbase + r * BB, BB)],
                                 dst_v.at[p, r]))
            return cps

        def _idx_wait(j):
            sb = j // SBB
            r = j % SBB

            @pl.when((r == 0) & (sb > 0) & (sb < NSB))
            def _wi():
                p = sb % 2
                for cp in _idx_rows(
                        sb, p, lambda a, b: pltpu.make_async_copy(a, b, isem)):
                    cp.wait()

        def _idx_prefetch(j):
            sb = j // SBB
            r = j % SBB

            @pl.when((r == 2) & (sb + 1 < NSB))
            def _pi():
                pn = (sb + 1) % 2
                _idx_rows(sb + 1, pn,
                          lambda a, b: pltpu.async_copy(a, b, isem))

        def _z(i, _):
            for l in range(8):
                rows_v[0, i, pl.ds(l * 16, 16)] = jnp.zeros((16,), jnp.float32)
            return _
        lax.fori_loop(0, BB, _z, None)
        zcps = []
        for j in range(16):
            r = BB if j < 15 else 32
            zcps.append(pltpu.async_copy(
                rows_v.at[0].at[pl.ds(0, r)],
                acc.at[pl.ds(s * 632 + j * BB, r)], gsem0))
        for cp in zcps:
            cp.wait()

        for r in range(SBB):
            pltpu.sync_copy(
                src_hbm.at[pl.ds(q * E + s * (E // 16) + r * BB, BB)],
                src_v.at[0, r])
            pltpu.sync_copy(
                dst_hbm.at[pl.ds(s * (E // 16) + r * BB, BB)],
                dst_v.at[0, r])
        plsc.subcore_barrier()

        _gath(0, 0, gsem0)

        def _pair(g, _):
            j0 = 2 * g
            j1 = j0 + 1
            j2 = j0 + 2

            @pl.when(g > 0)
            def _w1():
                _scat_wait(1, ssem1)
            _idx_wait(j1)
            _idx_prefetch(j1)
            _gath(j1, 1, gsem1)
            _gath_wait(j0, 0, gsem0)
            _scat(j0, 0, ssem0)
            _idx_wait(j2)
            _idx_prefetch(j2)
            _scat_wait(0, ssem0)

            @pl.when(j2 < NB)
            def _g2():
                _gath(j2, 0, gsem0)
            _gath_wait(j1, 1, gsem1)
            _scat(j1, 1, ssem1)
            return _
        lax.fori_loop(0, NB // 2, _pair, None)
        _scat_wait(1, ssem1)
        plsc.subcore_barrier()

        def _cpr(j):
            return BB if j < 15 else 32

        def _cin(j, fn, sem):
            r = _cpr(j)
            return fn(acc.at[pl.ds(s * 632 + j * BB, r)],
                      rows_v.at[j % 2].at[pl.ds(0, r)], sem)

        def _cout(j, fn, sem):
            r = _cpr(j)
            base = s * 632 + j * BB
            return fn(rows_v.at[j % 2].at[pl.ds(0, r)],
                      out_hbm.at[q].at[pl.ds(base, r)], sem)

        _cin(0, pltpu.async_copy, gsem0)
        for j in range(16):
            _cin(j, pltpu.make_async_copy, gsem0).wait()
            _cout(j, pltpu.async_copy, gsem1)
            if j + 1 < 16:
                if j >= 1:
                    _cout(j - 1, pltpu.make_async_copy, gsem1).wait()
                _cin(j + 1, pltpu.async_copy, gsem0)
        _cout(14, pltpu.make_async_copy, gsem1).wait()
        _cout(15, pltpu.make_async_copy, gsem1).wait()
        plsc.subcore_barrier()


_prop_kernel = pl.kernel(
    _prop_body,
    out_type=jax.ShapeDtypeStruct((4, ACCN, 128), jnp.float32),
    mesh=_MESH,
    scratch_types=[
        pltpu.VMEM((2, SBB, BB), jnp.int32),
        pltpu.VMEM((2, SBB, BB), jnp.int32),
        pltpu.VMEM((2, BB, 128), jnp.float32),
        pltpu.VMEM_SHARED((ACCN, 128), jnp.float32),
        pltpu.SemaphoreType.DMA,
        pltpu.SemaphoreType.DMA,
        pltpu.SemaphoreType.DMA,
        pltpu.SemaphoreType.DMA,
        pltpu.SemaphoreType.DMA,
    ],
)


def _deg_body(dst_hbm, out_hbm, dst_v, ones_v, zbuf, acc, ssem0, ssem1, isem):
    c = lax.axis_index("c")
    s = lax.axis_index("s")
    EPC = E // 2
    DPT = EPC // 16
    DNB = DPT // BB
    DNSB = DNB // SBB

    def _fill(ref, val):
        def _f(i, _):
            for l in range(8):
                ref[i, pl.ds(l * 16, 16)] = jnp.full((16,), val, jnp.float32)
            return _
        lax.fori_loop(0, BB, _f, None)

    _fill(ones_v, 1.0)
    _fill(zbuf, 0.0)
    for j in range(16):
        r = BB if j < 15 else 32
        pltpu.sync_copy(zbuf.at[pl.ds(0, r)],
                        acc.at[pl.ds(s * 632 + j * BB, r)])

    def _scat(j, sem):
        p = (j // SBB) % 2
        r = j % SBB
        return pltpu.async_copy(
            ones_v, acc.at[dst_v.at[p, r]], sem, add=True)

    def _scat_wait(sem):
        pltpu.make_async_copy(ones_v, acc.at[dst_v.at[0, 0]], sem).wait()

    def _idx_rows(sb, p, start):
        dbase = c * EPC + s * DPT + sb * (SBB * BB)
        return [start(dst_hbm.at[pl.ds(dbase + r * BB, BB)], dst_v.at[p, r])
                for r in range(SBB)]

    def _idx_wait(j):
        sb = j // SBB
        r = j % SBB

        @pl.when((r == 0) & (sb > 0) & (sb < DNSB))
        def _wi():
            for cp in _idx_rows(
                    sb, sb % 2,
                    lambda a, b: pltpu.make_async_copy(a, b, isem)):
                cp.wait()

    def _idx_prefetch(j):
        sb = j // SBB
        r = j % SBB

        @pl.when((r == 4) & (sb + 1 < DNSB))
        def _pi():
            _idx_rows(sb + 1, (sb + 1) % 2,
                      lambda a, b: pltpu.async_copy(a, b, isem))

    for r in range(SBB):
        pltpu.sync_copy(dst_hbm.at[pl.ds(c * EPC + s * DPT + r * BB, BB)],
                        dst_v.at[0, r])
    plsc.subcore_barrier()

    def _pair(g, _):
        j0 = 2 * g
        j1 = j0 + 1

        @pl.when(g > 1)
        def _w0():
            _scat_wait(ssem0)
        _idx_wait(j0)
        _idx_prefetch(j0)
        _scat(j0, ssem0)

        @pl.when(g > 1)
        def _w1():
            _scat_wait(ssem1)
        _idx_wait(j1)
        _idx_prefetch(j1)
        _scat(j1, ssem1)
        return _
    lax.fori_loop(0, DNB // 2, _pair, None)
    _idx_wait(DNB - 1)
    _scat(DNB - 1, ssem0)
    for _ in range(3):
        _scat_wait(ssem0)
    for _ in range(2):
        _scat_wait(ssem1)
    plsc.subcore_barrier()

    for j in range(16):
        r = BB if j < 15 else 32
        base = s * 632 + j * BB
        pltpu.sync_copy(acc.at[pl.ds(base, r)], zbuf.at[pl.ds(0, r)])
        pltpu.sync_copy(zbuf.at[pl.ds(0, r)],
                        out_hbm.at[c].at[pl.ds(base, r)])


_deg_kernel = pl.kernel(
    _deg_body,
    out_type=jax.ShapeDtypeStruct((2, ACCN, 128), jnp.float32),
    mesh=_MESH,
    scratch_types=[
        pltpu.VMEM((2, SBB, BB), jnp.int32),
        pltpu.VMEM((BB, 128), jnp.float32),
        pltpu.VMEM((BB, 128), jnp.float32),
        pltpu.VMEM_SHARED((ACCN, 128), jnp.float32),
        pltpu.SemaphoreType.DMA,
        pltpu.SemaphoreType.DMA,
        pltpu.SemaphoreType.DMA,
    ],
)


def _prep_edges_body(src_ref, dst_ref, deff_ref, seff_ref, srcq_ref):
    qq = pl.program_id(0)
    srcv = src_ref[...]
    dstv = dst_ref[...]
    loop = srcv == dstv
    deff_ref[...] = jnp.where(loop, TRASH, dstv)
    seff_ref[...] = jnp.where(loop, TRASH, srcv)
    srcq_ref[0] = srcv + qq * NPAD


def _prep_edges(src2, dst2):
    return pl.pallas_call(
        _prep_edges_body,
        grid=(4,),
        in_specs=[
            pl.BlockSpec((1250, 128), lambda qq: (0, 0)),
            pl.BlockSpec((1250, 128), lambda qq: (0, 0)),
        ],
        out_specs=(
            pl.BlockSpec((1250, 128), lambda qq: (0, 0)),
            pl.BlockSpec((1250, 128), lambda qq: (0, 0)),
            pl.BlockSpec((1, 1250, 128), lambda qq: (qq, 0, 0)),
        ),
        out_shape=(jax.ShapeDtypeStruct((1250, 128), jnp.int32),
                   jax.ShapeDtypeStruct((1250, 128), jnp.int32),
                   jax.ShapeDtypeStruct((4, 1250, 128), jnp.int32)),
    )(src2, dst2)


def _prep_nodes_body(degp_ref, f1_ref, f2_ref, c0_ref, u0_ref, dis_ref):
    deg = degp_ref[0, :, 0:1] + degp_ref[1, :, 0:1]
    dis = jnp.where(deg > 0, lax.rsqrt(jnp.maximum(deg, 1e-12)), 0.0)
    dis_ref[...] = dis
    f1 = f1_ref[...]
    f2 = f2_ref[...]
    planes = jnp.stack([f1[:, :128], f1[:, 128:], f2[:, :128], f2[:, 128:]])
    c0_ref[...] = planes
    u0_ref[...] = dis[None] * planes


def _prep_nodes(degp, feat, feat_):
    return pl.pallas_call(
        _prep_nodes_body,
        grid=(NMB,),
        in_specs=[
            pl.BlockSpec((2, MB, 128), lambda m: (0, m, 0)),
            pl.BlockSpec((MB, D), lambda m: (m, 0)),
            pl.BlockSpec((MB, D), lambda m: (m, 0)),
        ],
        out_specs=(
            pl.BlockSpec((4, MB, 128), lambda m: (0, m, 0)),
            pl.BlockSpec((4, MB, 128), lambda m: (0, m, 0)),
            pl.BlockSpec((MB, 1), lambda m: (m, 0)),
        ),
        out_shape=(jax.ShapeDtypeStruct((4, NPAD, 128), jnp.float32),
                   jax.ShapeDtypeStruct((4, NPAD, 128), jnp.float32),
                   jax.ShapeDtypeStruct((NPAD, 1), jnp.float32)),
    )(degp, feat, feat_)


def _stage1_body(r1_ref, dis_ref, o_ref):
    dis = dis_ref[...]
    o_ref[...] = (-dis * dis)[None] * r1_ref[...]


def _stage1(r1, dis2):
    return pl.pallas_call(
        _stage1_body,
        grid=(4, NMB),
        in_specs=[
            pl.BlockSpec((1, MB, 128), lambda q, m: (q, m, 0)),
            pl.BlockSpec((MB, 1), lambda q, m: (m, 0)),
        ],
        out_specs=pl.BlockSpec((1, MB, 128), lambda q, m: (q, m, 0)),
        out_shape=jax.ShapeDtypeStruct((4, NPAD, 128), jnp.float32),
    )(r1.reshape(4, ACCN, 128), dis2)


def _stage2_body(r2_ref, u0_ref, dis_ref, o_ref):
    dis = dis_ref[...]
    o_ref[...] = (-2.0 * dis * dis)[None] * r2_ref[...] - u0_ref[...]


def _stage2(r2, u0, dis2):
    return pl.pallas_call(
        _stage2_body,
        grid=(4, NMB),
        in_specs=[
            pl.BlockSpec((1, MB, 128), lambda q, m: (q, m, 0)),
            pl.BlockSpec((1, MB, 128), lambda q, m: (q, m, 0)),
            pl.BlockSpec((MB, 1), lambda q, m: (m, 0)),
        ],
        out_specs=pl.BlockSpec((1, MB, 128), lambda q, m: (q, m, 0)),
        out_shape=jax.ShapeDtypeStruct((4, NPAD, 128), jnp.float32),
    )(r2.reshape(4, ACCN, 128), u0, dis2)


def _matmul_body(c0_ref, r1_ref, r2_ref, r3_ref, dis_ref, w_ref, b_ref, o_ref):
    dis = dis_ref[...]
    cat = lambda a: jnp.concatenate([a[0], a[1]], axis=1)
    tx0 = cat(c0_ref[...])
    r1 = cat(r1_ref[...])
    tx1 = -dis * r1
    tx2 = -2.0 * dis * cat(r2_ref[...]) - tx0
    tx3 = -2.0 * dis * cat(r3_ref[...]) - tx1
    w = w_ref[0]
    acc = jnp.dot(tx0, w[0], preferred_element_type=jnp.float32)
    acc += jnp.dot(tx1, w[1], preferred_element_type=jnp.float32)
    acc += jnp.dot(tx2, w[2], preferred_element_type=jnp.float32)
    acc += jnp.dot(tx3, w[3], preferred_element_type=jnp.float32)
    o_ref[0] = acc + b_ref[0]


def _cheb_matmul(c0, r1, r2, r3, dis2, wl, bl):
    pspec = pl.BlockSpec((2, MB, 128), lambda b, m: (b, m, 0))
    return pl.pallas_call(
        _matmul_body,
        grid=(2, NMB),
        in_specs=[
            pspec, pspec, pspec, pspec,
            pl.BlockSpec((MB, 1), lambda b, m: (m, 0)),
            pl.BlockSpec((1, 4, D, D), lambda b, m: (b, 0, 0, 0)),
            pl.BlockSpec((1, 1, D), lambda b, m: (b, 0, 0)),
        ],
        out_specs=pl.BlockSpec((1, MB, D), lambda b, m: (b, m, 0)),
        out_shape=jax.ShapeDtypeStruct((2, N, D), jnp.float32),
    )(c0, r1.reshape(4, ACCN, 128), r2.reshape(4, ACCN, 128),
      r3.reshape(4, ACCN, 128), dis2, wl, bl)


def _stats_body(conv_ref, gms_ref, o_ref):
    m = pl.program_id(1)
    x = conv_ref[0]
    s1 = jnp.sum(x, axis=0, keepdims=True)
    s2 = jnp.sum(x * x, axis=0, keepdims=True)
    blk = jnp.concatenate([s1, s2, jnp.zeros((6, D), jnp.float32)], 0)[None]

    @pl.when(m == 0)
    def _i():
        o_ref[...] = blk

    @pl.when(m > 0)
    def _a():
        o_ref[...] += blk

    @pl.when(m == NMB - 1)
    def _f():
        tot = o_ref[...]
        mean = tot[0, 0:1, :] / N
        ex2 = tot[0, 1:2, :] / N
        shift = mean * gms_ref[0]
        var = ex2 - 2.0 * shift * mean + shift * shift
        inv = lax.rsqrt(var + EPS)
        o_ref[...] = jnp.concatenate(
            [shift, inv, jnp.zeros((6, D), jnp.float32)], 0)[None]


def _stats(conv, gms):
    return pl.pallas_call(
        _stats_body,
        grid=(2, NMB),
        in_specs=[
            pl.BlockSpec((1, MB, D), lambda b, m: (b, m, 0)),
            pl.BlockSpec((1, 1, D), lambda b, m: (b, 0, 0)),
        ],
        out_specs=pl.BlockSpec((1, 8, D), lambda b, m: (b, 0, 0)),
        out_shape=jax.ShapeDtypeStruct((2, 8, D), jnp.float32),
    )(conv, gms)


def _norm_body(conv_ref, st_ref, g_ref, b_ref, dis_ref, c0_ref, u_ref):
    x = conv_ref[0]
    shift = st_ref[0, 0:1, :]
    inv = st_ref[0, 1:2, :]
    y = (x - shift) * inv * g_ref[0] + b_ref[0]
    h = jnp.where(y >= 0, y, 0.1 * y)
    c0_ref[0] = h
    u_ref[0] = dis_ref[...] * h


def _normalize(conv, st, gg, gb, dis2):
    return pl.pallas_call(
        _norm_body,
        grid=(4, NMB),
        in_specs=[
            pl.BlockSpec((1, MB, 128), lambda q, m: (q // 2, m, q % 2)),
            pl.BlockSpec((1, 8, 128), lambda q, m: (q // 2, 0, q % 2)),
            pl.BlockSpec((1, 1, 128), lambda q, m: (q // 2, 0, q % 2)),
            pl.BlockSpec((1, 1, 128), lambda q, m: (q // 2, 0, q % 2)),
            pl.BlockSpec((MB, 1), lambda q, m: (m, 0)),
        ],
        out_specs=(
            pl.BlockSpec((1, MB, 128), lambda q, m: (q, m, 0)),
            pl.BlockSpec((1, MB, 128), lambda q, m: (q, m, 0)),
        ),
        out_shape=(jax.ShapeDtypeStruct((4, NPAD, 128), jnp.float32),
                   jax.ShapeDtypeStruct((4, NPAD, 128), jnp.float32)),
    )(conv, st, gg, gb, dis2)


def _final_body(conv_ref, st_ref, g_ref, b_ref, feat_ref, o_ref):
    m = pl.program_id(1)
    x = conv_ref[0]
    shift = st_ref[0, 0:1, :]
    inv = st_ref[0, 1:2, :]
    y = feat_ref[0] + (x - shift) * inv * g_ref[0] + b_ref[0]
    h = jnp.maximum(y, 0.0)
    blk = jnp.concatenate(
        [jnp.sum(h, axis=0, keepdims=True), jnp.zeros((7, D), jnp.float32)],
        0)[None]

    @pl.when(m == 0)
    def _i():
        o_ref[...] = blk

    @pl.when(m > 0)
    def _a():
        o_ref[...] += blk


def _final_phase(conv, st, gg, gb, feats):
    return pl.pallas_call(
        _final_body,
        grid=(2, NMB),
        in_specs=[
            pl.BlockSpec((1, MB, D), lambda b, m: (b, m, 0)),
            pl.BlockSpec((1, 8, D), lambda b, m: (b, 0, 0)),
            pl.BlockSpec((1, 1, D), lambda b, m: (b, 0, 0)),
            pl.BlockSpec((1, 1, D), lambda b, m: (b, 0, 0)),
            pl.BlockSpec((1, MB, D), lambda b, m: (b, m, 0)),
        ],
        out_specs=pl.BlockSpec((1, 8, D), lambda b, m: (b, 0, 0)),
        out_shape=jax.ShapeDtypeStruct((2, 8, D), jnp.float32),
    )(conv, st, gg, gb, feats)


def _head_body(cs_ref, lw_ref, lb_ref, o_ref):
    hm = jnp.maximum(cs_ref[0, 0:1, :] / N, 0.0)
    z = jnp.dot(hm, lw_ref[0], preferred_element_type=jnp.float32)
    z = z + lb_ref[0]
    z = z - jnp.max(z, axis=-1, keepdims=True)
    e = jnp.exp(z)
    o_ref[0] = e / jnp.sum(e, axis=-1, keepdims=True)


def _head(colsum, lwc, lbc):
    return pl.pallas_call(
        _head_body,
        grid=(2,),
        in_specs=[
            pl.BlockSpec((1, 8, D), lambda b: (b, 0, 0)),
            pl.BlockSpec((1, D, OUT), lambda b: (b, 0, 0)),
            pl.BlockSpec((1, 1, OUT), lambda b: (b, 0, 0)),
        ],
        out_specs=pl.BlockSpec((1, 1, OUT), lambda b: (b, 0, 0)),
        out_shape=jax.ShapeDtypeStruct((2, 1, OUT), jnp.float32),
    )(colsum, lwc, lbc)


def kernel(edge_index, feat, feat_, W1, b1, gn1_g, gn1_b, gn1_ms, lin1_w, lin1_b,
           W2, b2, gn2_g, gn2_b, gn2_ms, lin2_w, lin2_b):
    src, dst = edge_index[0], edge_index[1]
    src2 = src.reshape(1250, 128)
    dst2 = dst.reshape(1250, 128)
    deff2, seff2, srcq = _prep_edges(src2, dst2)

    src4 = srcq.reshape(4 * E)
    dst4 = deff2.reshape(E)

    degp = _deg_kernel(seff2.reshape(E))
    c0, u, dis2 = _prep_nodes(degp, feat, feat_)

    Wc = jnp.stack([W1, W2])
    bc = jnp.stack([b1, b2]).reshape(2, 4, 1, D)
    gc = jnp.stack([gn1_g, gn2_g]).reshape(2, 4, 1, D)
    bgc = jnp.stack([gn1_b, gn2_b]).reshape(2, 4, 1, D)
    msc = jnp.stack([gn1_ms, gn2_ms]).reshape(2, 4, 1, D)
    feats = jnp.stack([feat, feat_])
    lwc = jnp.stack([lin1_w, lin2_w])
    lbc = jnp.stack([lin1_b, lin2_b]).reshape(2, 1, OUT)

    for i in range(4):
        r1 = _prop_kernel(u.reshape(4 * NPAD, 128), src4, dst4)
        in2 = _stage1(r1, dis2)
        r2 = _prop_kernel(in2.reshape(4 * NPAD, 128), src4, dst4)
        in3 = _stage2(r2, u, dis2)
        r3 = _prop_kernel(in3.reshape(4 * NPAD, 128), src4, dst4)
        conv = _cheb_matmul(c0, r1, r2, r3, dis2, Wc[:, i], bc[:, i])
        st = _stats(conv, msc[:, i])
        if i < 3:
            c0, u = _normalize(conv, st, gc[:, i], bgc[:, i], dis2)
        else:
            colsum = _final_phase(conv, st, gc[:, i], bgc[:, i], feats)

    out = _head(colsum, lwc, lbc)
    return (out[0, 0], out[1, 0])
